# baseline (speedup 1.0000x reference)
import functools

import jax
import jax.numpy as jnp
from jax import lax
from jax.experimental import pallas as pl
from jax.experimental.pallas import tpu as pltpu
from jax.experimental.pallas import tpu_sc as plsc

BATCH = 16384
K = 128
LANES = 16
NC = 2
NS = 16
NW = NC * NS
ROWS_PER_W = BATCH // NW
CHUNK = 128
NCHUNK = ROWS_PER_W // CHUNK

_mesh = plsc.VectorSubcoreMesh(core_axis_name="c", subcore_axis_name="s")


@functools.partial(
    pl.kernel,
    out_type=[
        jax.ShapeDtypeStruct((BATCH,), jnp.float32),
        jax.ShapeDtypeStruct((BATCH, K), jnp.float32),
        jax.ShapeDtypeStruct((BATCH, K), jnp.float32),
    ],
    mesh=_mesh,
    scratch_types=[
        pltpu.VMEM((CHUNK,), jnp.int32),
        pltpu.VMEM((CHUNK,), jnp.int32),
        pltpu.VMEM((CHUNK, K), jnp.float32),
        pltpu.VMEM((CHUNK, K), jnp.float32),
        pltpu.VMEM((ROWS_PER_W,), jnp.float32),
        pltpu.SemaphoreType.DMA,
        pltpu.SemaphoreType.DMA,
    ],
    compiler_params=pltpu.CompilerParams(needs_layout_passes=False),
)
def _sc_body(users_hbm, items_hbm, gu_hbm, gi_hbm,
             xui_hbm, gu_out_hbm, gi_out_hbm,
             idxu_v, idxi_v, u_buf, i_buf, xui_v, sem_u, sem_i):
    wid = lax.axis_index("s") * NC + lax.axis_index("c")
    base = wid * ROWS_PER_W

    def chunk_body(chunk, _):
        co = chunk * CHUNK
        pltpu.sync_copy(users_hbm.at[pl.ds(base + co, CHUNK)], idxu_v)
        pltpu.sync_copy(items_hbm.at[pl.ds(base + co, CHUNK)], idxi_v)
        cp_u = pltpu.async_copy(gu_hbm.at[idxu_v], u_buf, sem_u)
        cp_i = pltpu.async_copy(gi_hbm.at[idxi_v], i_buf, sem_i)
        cp_u.wait()
        cp_i.wait()

        lane = lax.iota(jnp.int32, LANES)

        def group_body(g, _):
            base_r = g * LANES
            out16 = jnp.zeros((LANES,), jnp.float32)
            for j in range(LANES):
                ur = u_buf.at[base_r + j]
                ir = i_buf.at[base_r + j]
                acc = ur[pl.ds(0, LANES)] * ir[pl.ds(0, LANES)]
                for k in range(1, K // LANES):
                    acc = acc + ur[pl.ds(k * LANES, LANES)] * ir[pl.ds(k * LANES, LANES)]
                s = jnp.sum(acc)
                out16 = out16 + jnp.where(lane == j, s, 0.0)
            xui_v[pl.ds(co + base_r, LANES)] = out16
            return 0

        lax.fori_loop(0, CHUNK // LANES, group_body, 0)

        pltpu.sync_copy(u_buf, gu_out_hbm.at[pl.ds(base + co, CHUNK)])
        pltpu.sync_copy(i_buf, gi_out_hbm.at[pl.ds(base + co, CHUNK)])
        return 0

    lax.fori_loop(0, NCHUNK, chunk_body, 0)
    pltpu.sync_copy(xui_v, xui_hbm.at[pl.ds(base, ROWS_PER_W)])


def kernel(users, items, Gu, Gi):
    xui, gu, gi = _sc_body(users.astype(jnp.int32), items.astype(jnp.int32), Gu, Gi)
    return xui, gu, gi


# speedup vs baseline: 1.0973x; 1.0973x over previous
import functools

import jax
import jax.numpy as jnp
from jax import lax
from jax.experimental import pallas as pl
from jax.experimental.pallas import tpu as pltpu
from jax.experimental.pallas import tpu_sc as plsc

BATCH = 16384
K = 128
LANES = 16
NC = 2
NS = 16
NW = NC * NS
ROWS_PER_W = BATCH // NW
CHUNK = 128
NCHUNK = ROWS_PER_W // CHUNK
NSLOT = 3

_mesh = plsc.VectorSubcoreMesh(core_axis_name="c", subcore_axis_name="s")


@functools.partial(
    pl.kernel,
    out_type=[
        jax.ShapeDtypeStruct((BATCH,), jnp.float32),
        jax.ShapeDtypeStruct((BATCH, K), jnp.float32),
        jax.ShapeDtypeStruct((BATCH, K), jnp.float32),
    ],
    mesh=_mesh,
    scratch_types=[
        pltpu.VMEM((ROWS_PER_W,), jnp.int32),
        pltpu.VMEM((ROWS_PER_W,), jnp.int32),
        pltpu.VMEM((CHUNK, K), jnp.float32),
        pltpu.VMEM((CHUNK, K), jnp.float32),
        pltpu.VMEM((CHUNK, K), jnp.float32),
        pltpu.VMEM((CHUNK, K), jnp.float32),
        pltpu.VMEM((CHUNK, K), jnp.float32),
        pltpu.VMEM((CHUNK, K), jnp.float32),
        pltpu.VMEM((ROWS_PER_W,), jnp.float32),
        pltpu.SemaphoreType.DMA,
        pltpu.SemaphoreType.DMA,
        pltpu.SemaphoreType.DMA,
        pltpu.SemaphoreType.DMA,
        pltpu.SemaphoreType.DMA,
        pltpu.SemaphoreType.DMA,
    ],
    compiler_params=pltpu.CompilerParams(needs_layout_passes=False),
)
def _sc_body(users_hbm, items_hbm, gu_hbm, gi_hbm,
             xui_hbm, gu_out_hbm, gi_out_hbm,
             idxu_v, idxi_v, ub0, ub1, ub2, ib0, ib1, ib2, xui_v,
             sg0, sg1, sg2, sw0, sw1, sw2):
    ubufs = (ub0, ub1, ub2)
    ibufs = (ib0, ib1, ib2)
    sg = (sg0, sg1, sg2)
    sw = (sw0, sw1, sw2)

    wid = lax.axis_index("s") * NC + lax.axis_index("c")
    base = wid * ROWS_PER_W

    pltpu.sync_copy(users_hbm.at[pl.ds(base, ROWS_PER_W)], idxu_v)
    pltpu.sync_copy(items_hbm.at[pl.ds(base, ROWS_PER_W)], idxi_v)

    lane = lax.iota(jnp.int32, LANES)

    def fire_gather(n, s):
        co = n * CHUNK
        cu = pltpu.async_copy(gu_hbm.at[idxu_v.at[pl.ds(co, CHUNK)]], ubufs[s], sg[s])
        ci = pltpu.async_copy(gi_hbm.at[idxi_v.at[pl.ds(co, CHUNK)]], ibufs[s], sg[s])
        return cu, ci

    def fire_writeback(n, s):
        co = n * CHUNK
        wu = pltpu.async_copy(ubufs[s], gu_out_hbm.at[pl.ds(base + co, CHUNK)], sw[s])
        wi = pltpu.async_copy(ibufs[s], gi_out_hbm.at[pl.ds(base + co, CHUNK)], sw[s])
        return wu, wi

    def dot_chunk(n, s):
        co = n * CHUNK
        u_buf, i_buf = ubufs[s], ibufs[s]

        def group_body(g, _):
            base_r = g * LANES
            out16 = jnp.zeros((LANES,), jnp.float32)
            for j in range(LANES):
                ur = u_buf.at[base_r + j]
                ir = i_buf.at[base_r + j]
                acc = ur[pl.ds(0, LANES)] * ir[pl.ds(0, LANES)]
                for k in range(1, K // LANES):
                    acc = acc + ur[pl.ds(k * LANES, LANES)] * ir[pl.ds(k * LANES, LANES)]
                s_j = jnp.sum(acc)
                out16 = out16 + jnp.where(lane == j, s_j, 0.0)
            xui_v[pl.ds(co + base_r, LANES)] = out16
            return 0

        lax.fori_loop(0, CHUNK // LANES, group_body, 0)

    cp = {n: fire_gather(n, n) for n in range(NSLOT)}
    wb = {}

    for c in cp[0]:
        c.wait()
    dot_chunk(0, 0)
    wb[0] = fire_writeback(0, 0)

    for c in cp[1]:
        c.wait()
    dot_chunk(1, 1)
    wb[1] = fire_writeback(1, 1)

    for c in wb[0]:
        c.wait()
    cp[3] = fire_gather(3, 0)

    for c in cp[2]:
        c.wait()
    dot_chunk(2, 2)
    wb[2] = fire_writeback(2, 2)

    for c in cp[3]:
        c.wait()
    dot_chunk(3, 0)
    wb[3] = fire_writeback(3, 0)

    for n in (1, 2, 3):
        for c in wb[n]:
            c.wait()

    pltpu.sync_copy(xui_v, xui_hbm.at[pl.ds(base, ROWS_PER_W)])


def kernel(users, items, Gu, Gi):
    xui, gu, gi = _sc_body(users.astype(jnp.int32), items.astype(jnp.int32), Gu, Gi)
    return xui, gu, gi
